# Initial kernel scaffold; baseline (speedup 1.0000x reference)
import jax
import jax.numpy as jnp
from jax.experimental import pallas as pl


def kernel(positions, tables, W1, b1, W2, b2, W3, b3):
    raise NotImplementedError("write your pallas kernel here")


# speedup vs baseline: 2.0047x; 2.0047x over previous
import functools

import jax
import jax.numpy as jnp
from jax import lax
from jax.experimental import pallas as pl
from jax.experimental.pallas import tpu as pltpu
from jax.experimental.pallas import tpu_sc as plsc

NUM_LEVELS = 16
F_PER_LEVEL = 2
LOG2_HASHMAP = 19
TABLE_SIZE = 2 ** LOG2_HASHMAP
BATCH = 262144
HIDDEN = 64
MLP_IN = NUM_LEVELS * F_PER_LEVEL

_RES = [int(16 * (2048 / 16) ** (l / (NUM_LEVELS - 1))) for l in range(NUM_LEVELS)]
_M1 = 2654435761 - (1 << 32)
_M2 = 805459861

_NW = 32
_CHUNK = 4096


def _hash_body(pos_ref, res_ref, idx_ref):
    p = pos_ref[...]
    res = res_ref[...]
    c0 = (p[:, 0:1] * res).astype(jnp.int32)
    c1 = (p[:, 1:2] * res).astype(jnp.int32)
    c2 = (p[:, 2:3] * res).astype(jnp.int32)
    h = c0 ^ (c1 * jnp.int32(_M1)) ^ (c2 * jnp.int32(_M2))
    level_off = lax.broadcasted_iota(jnp.int32, (1, NUM_LEVELS), 1) * jnp.int32(TABLE_SIZE)
    idx_ref[...] = (h & jnp.int32(TABLE_SIZE - 1)) + level_off


def _compute_indices(positions):
    blk = 8192
    grid = BATCH // blk
    res = jnp.array(_RES, dtype=jnp.float32).reshape(1, NUM_LEVELS)
    return pl.pallas_call(
        _hash_body,
        grid=(grid,),
        in_specs=[
            pl.BlockSpec((blk, 3), lambda i: (i, i * 0)),
            pl.BlockSpec((1, NUM_LEVELS), lambda i: (i * 0, i * 0)),
        ],
        out_specs=pl.BlockSpec((blk, NUM_LEVELS), lambda i: (i, i * 0)),
        out_shape=jax.ShapeDtypeStruct((BATCH, NUM_LEVELS), jnp.int32),
    )(positions, res)


def _gather_sc(flat_table, idx_flat):
    n = BATCH * NUM_LEVELS
    per_w = n // _NW
    n_ch = per_w // _CHUNK
    mesh = plsc.VectorSubcoreMesh(core_axis_name="c", subcore_axis_name="s")

    @functools.partial(
        pl.kernel,
        out_type=jax.ShapeDtypeStruct((n, F_PER_LEVEL), jnp.float32),
        mesh=mesh,
        compiler_params=pltpu.CompilerParams(use_tc_tiling_on_sc=False),
        scratch_types=[
            pltpu.VMEM((_CHUNK,), jnp.int32),
            pltpu.VMEM((_CHUNK, F_PER_LEVEL), jnp.float32),
            pltpu.SemaphoreType.DMA,
        ],
    )
    def k(table_hbm, idx_hbm, out_hbm, idx_v, rows_v, gsem):
        wid = (lax.axis_index("s") * 2 + lax.axis_index("c")).astype(jnp.int32)
        base = wid * jnp.int32(per_w)

        @pl.loop(jnp.int32(0), jnp.int32(n_ch))
        def body(ch):
            off = base + ch * jnp.int32(_CHUNK)
            pltpu.sync_copy(idx_hbm.at[pl.ds(off, _CHUNK)], idx_v)
            pltpu.async_copy(table_hbm.at[idx_v], rows_v, gsem).wait()
            pltpu.sync_copy(rows_v, out_hbm.at[pl.ds(off, _CHUNK)])

    return k(flat_table, idx_flat)


def _mlp_body(x_ref, w1_ref, b1_ref, w2_ref, b2_ref, w3_ref, b3_ref, o_ref):
    x = x_ref[...]
    h = jnp.dot(x, w1_ref[...], preferred_element_type=jnp.float32) + b1_ref[...]
    h = jnp.maximum(h, 0.0)
    h = jnp.dot(h, w2_ref[...], preferred_element_type=jnp.float32) + b2_ref[...]
    h = jnp.maximum(h, 0.0)
    o = jnp.dot(h, w3_ref[...], preferred_element_type=jnp.float32) + b3_ref[...]
    sig = 1.0 / (1.0 + jnp.exp(-o))
    sp = jnp.maximum(o, 0.0) + jnp.log(1.0 + jnp.exp(-jnp.abs(o)))
    col = lax.broadcasted_iota(jnp.int32, o.shape, 1)
    o_ref[...] = jnp.where(col < 3, sig, sp)


def _mlp(feats, W1, b1, W2, b2, W3, b3):
    blk = 4096
    grid = BATCH // blk
    full = lambda a: pl.BlockSpec(a.shape, lambda i: tuple(i * 0 for _ in a.shape))
    return pl.pallas_call(
        _mlp_body,
        grid=(grid,),
        in_specs=[
            pl.BlockSpec((blk, MLP_IN), lambda i: (i, i * 0)),
            full(W1), full(b1), full(W2), full(b2), full(W3), full(b3),
        ],
        out_specs=pl.BlockSpec((blk, 4), lambda i: (i, i * 0)),
        out_shape=jax.ShapeDtypeStruct((BATCH, 4), jnp.float32),
    )(feats, W1, b1, W2, b2, W3, b3)


def kernel(positions, tables, W1, b1, W2, b2, W3, b3):
    f32 = jnp.float32
    positions = positions.astype(f32)
    tables = tables.astype(f32)
    W1, b1, W2, b2, W3, b3 = (a.astype(f32) for a in (W1, b1, W2, b2, W3, b3))
    idx = _compute_indices(positions)
    flat_table = tables.reshape(NUM_LEVELS * TABLE_SIZE, F_PER_LEVEL)
    rows = _gather_sc(flat_table, idx.reshape(-1))
    feats = rows.reshape(BATCH, MLP_IN)
    y = _mlp(
        feats,
        W1,
        b1.reshape(1, HIDDEN),
        W2,
        b2.reshape(1, HIDDEN),
        W3,
        b3.reshape(1, 4),
    )
    y = y.astype(jnp.float64)
    return y[:, :3], y[:, 3:4]


# speedup vs baseline: 58.7908x; 29.3269x over previous
import functools

import jax
import jax.numpy as jnp
from jax import lax
from jax.experimental import pallas as pl
from jax.experimental.pallas import tpu as pltpu
from jax.experimental.pallas import tpu_sc as plsc

NUM_LEVELS = 16
F_PER_LEVEL = 2
LOG2_HASHMAP = 19
TABLE_SIZE = 2 ** LOG2_HASHMAP
BATCH = 262144
HIDDEN = 64
MLP_IN = NUM_LEVELS * F_PER_LEVEL

_RES = [int(16 * (2048 / 16) ** (l / (NUM_LEVELS - 1))) for l in range(NUM_LEVELS)]
_M1 = 2654435761 - (1 << 32)
_M2 = 805459861

_NW = 32
_CHUNK = 8192


def _hash_body(pos_ref, res_ref, idx_ref):
    p = pos_ref[...]
    res = res_ref[...]
    r = lax.broadcasted_iota(jnp.int32, (32, 1), 0)
    lvl = r >> 1
    feat = r & 1
    c0 = (p[0:1, :] * res).astype(jnp.int32)
    c1 = (p[1:2, :] * res).astype(jnp.int32)
    c2 = (p[2:3, :] * res).astype(jnp.int32)
    h = c0 ^ (c1 * jnp.int32(_M1)) ^ (c2 * jnp.int32(_M2))
    e = h & jnp.int32(TABLE_SIZE - 1)
    idx_ref[...] = (
        (lvl << 20)
        + ((e >> 7) << 8)
        + (feat << 7)
        + (e & jnp.int32(127))
    )


def _compute_indices(pos_t):
    blk = 2048
    grid = BATCH // blk
    res = jnp.repeat(jnp.array(_RES, dtype=jnp.float32), 2).reshape(32, 1)
    return pl.pallas_call(
        _hash_body,
        grid=(grid,),
        in_specs=[
            pl.BlockSpec((3, blk), lambda i: (i * 0, i)),
            pl.BlockSpec((32, 1), lambda i: (i * 0, i * 0)),
        ],
        out_specs=pl.BlockSpec((32, blk), lambda i: (i * 0, i)),
        out_shape=jax.ShapeDtypeStruct((MLP_IN, BATCH), jnp.int32),
    )(pos_t, res)


def _gather_sc(table_1d, idx):
    per_w = BATCH // _NW
    n_ch = per_w // _CHUNK
    mesh = plsc.VectorSubcoreMesh(core_axis_name="c", subcore_axis_name="s")

    @functools.partial(
        pl.kernel,
        out_type=jax.ShapeDtypeStruct((MLP_IN, BATCH), jnp.float32),
        mesh=mesh,
        compiler_params=pltpu.CompilerParams(use_tc_tiling_on_sc=False),
        scratch_types=[
            pltpu.VMEM((_CHUNK,), jnp.int32),
            pltpu.VMEM((_CHUNK,), jnp.float32),
            pltpu.SemaphoreType.DMA,
        ],
    )
    def k(table_hbm, idx_hbm, out_hbm, idx_v, val_v, gsem):
        wid = (lax.axis_index("s") * 2 + lax.axis_index("c")).astype(jnp.int32)
        base = wid * jnp.int32(per_w)

        @pl.loop(jnp.int32(0), jnp.int32(MLP_IN * n_ch))
        def body(it):
            r = it // jnp.int32(n_ch)
            off = base + (it % jnp.int32(n_ch)) * jnp.int32(_CHUNK)
            pltpu.sync_copy(idx_hbm.at[r, pl.ds(off, _CHUNK)], idx_v)
            pltpu.async_copy(table_hbm.at[idx_v], val_v, gsem).wait()
            pltpu.sync_copy(val_v, out_hbm.at[r, pl.ds(off, _CHUNK)])

    return k(table_1d, idx)


def _mlp_body(x_ref, w1_ref, b1_ref, w2_ref, b2_ref, w3_ref, b3_ref, o_ref):
    x = x_ref[...]
    h = jnp.dot(w1_ref[...], x, preferred_element_type=jnp.float32) + b1_ref[...]
    h = jnp.maximum(h, 0.0)
    h = jnp.dot(w2_ref[...], h, preferred_element_type=jnp.float32) + b2_ref[...]
    h = jnp.maximum(h, 0.0)
    o = jnp.dot(w3_ref[...], h, preferred_element_type=jnp.float32) + b3_ref[...]
    sig = 1.0 / (1.0 + jnp.exp(-o))
    sp = jnp.maximum(o, 0.0) + jnp.log(1.0 + jnp.exp(-jnp.abs(o)))
    row = lax.broadcasted_iota(jnp.int32, o.shape, 0)
    o_ref[...] = jnp.where(row < 3, sig, sp)


def _mlp(feats, W1t, b1, W2t, b2, W3t, b3):
    blk = 2048
    grid = BATCH // blk
    full = lambda a: pl.BlockSpec(a.shape, lambda i: tuple(i * 0 for _ in a.shape))
    return pl.pallas_call(
        _mlp_body,
        grid=(grid,),
        in_specs=[
            pl.BlockSpec((MLP_IN, blk), lambda i: (i * 0, i)),
            full(W1t), full(b1), full(W2t), full(b2), full(W3t), full(b3),
        ],
        out_specs=pl.BlockSpec((4, blk), lambda i: (i * 0, i)),
        out_shape=jax.ShapeDtypeStruct((4, BATCH), jnp.float32),
    )(feats, W1t, b1, W2t, b2, W3t, b3)


def kernel(positions, tables, W1, b1, W2, b2, W3, b3):
    f32 = jnp.float32
    positions = positions.astype(f32)
    tables = tables.astype(f32)
    W1, b1, W2, b2, W3, b3 = (a.astype(f32) for a in (W1, b1, W2, b2, W3, b3))

    idx = _compute_indices(positions.T)
    table_1d = (
        tables.reshape(NUM_LEVELS, TABLE_SIZE // 128, 128, F_PER_LEVEL)
        .transpose(0, 1, 3, 2)
        .reshape(-1)
    )
    feats = _gather_sc(table_1d, idx)
    y = _mlp(
        feats,
        W1.T,
        b1.reshape(HIDDEN, 1),
        W2.T,
        b2.reshape(HIDDEN, 1),
        W3.T,
        b3.reshape(4, 1),
    )
    yt = y.T.astype(jnp.float64)
    return yt[:, :3], yt[:, 3:4]


# speedup vs baseline: 64.1758x; 1.0916x over previous
import functools

import jax
import jax.numpy as jnp
from jax import lax
from jax.experimental import pallas as pl
from jax.experimental.pallas import tpu as pltpu
from jax.experimental.pallas import tpu_sc as plsc

NUM_LEVELS = 16
F_PER_LEVEL = 2
LOG2_HASHMAP = 19
TABLE_SIZE = 2 ** LOG2_HASHMAP
BATCH = 262144
HIDDEN = 64
MLP_IN = NUM_LEVELS * F_PER_LEVEL

_RES = [int(16 * (2048 / 16) ** (l / (NUM_LEVELS - 1))) for l in range(NUM_LEVELS)]
_M1 = 2654435761 - (1 << 32)
_M2 = 805459861

_NW = 32
_CHUNK = 8192


def _hash_body(pos_ref, res_ref, idx_ref):
    p = pos_ref[...]
    res = res_ref[...]
    r = lax.broadcasted_iota(jnp.int32, (32, 1), 0)
    lvl = r >> 1
    feat = r & 1
    c0 = (p[0:1, :] * res).astype(jnp.int32)
    c1 = (p[1:2, :] * res).astype(jnp.int32)
    c2 = (p[2:3, :] * res).astype(jnp.int32)
    h = c0 ^ (c1 * jnp.int32(_M1)) ^ (c2 * jnp.int32(_M2))
    e = h & jnp.int32(TABLE_SIZE - 1)
    idx_ref[...] = (
        (lvl << 20)
        + ((e >> 7) << 8)
        + (feat << 7)
        + (e & jnp.int32(127))
    )


def _compute_indices(pos_t):
    blk = 2048
    grid = BATCH // blk
    res = jnp.repeat(jnp.array(_RES, dtype=jnp.float32), 2).reshape(32, 1)
    return pl.pallas_call(
        _hash_body,
        grid=(grid,),
        in_specs=[
            pl.BlockSpec((3, blk), lambda i: (i * 0, i)),
            pl.BlockSpec((32, 1), lambda i: (i * 0, i * 0)),
        ],
        out_specs=pl.BlockSpec((32, blk), lambda i: (i * 0, i)),
        out_shape=jax.ShapeDtypeStruct((MLP_IN, BATCH), jnp.int32),
    )(pos_t, res)


def _gather_sc(table_1d, idx):
    per_w = BATCH // _NW
    n_it = MLP_IN
    mesh = plsc.VectorSubcoreMesh(core_axis_name="c", subcore_axis_name="s")

    @functools.partial(
        pl.kernel,
        out_type=jax.ShapeDtypeStruct((MLP_IN, BATCH), jnp.float32),
        mesh=mesh,
        compiler_params=pltpu.CompilerParams(use_tc_tiling_on_sc=False),
        scratch_types=[
            pltpu.VMEM((per_w,), jnp.int32),
            pltpu.VMEM((per_w,), jnp.int32),
            pltpu.VMEM((per_w,), jnp.float32),
            pltpu.VMEM((per_w,), jnp.float32),
            pltpu.SemaphoreType.DMA,
            pltpu.SemaphoreType.DMA,
            pltpu.SemaphoreType.DMA,
            pltpu.SemaphoreType.DMA,
        ],
    )
    def k(table_hbm, idx_hbm, out_hbm, idx0, idx1, val0, val1, g0, g1, o0, o1):
        wid = (lax.axis_index("s") * 2 + lax.axis_index("c")).astype(jnp.int32)
        base = wid * jnp.int32(per_w)
        slots = ((idx0, val0, g0, o0), (idx1, val1, g1, o1))

        pltpu.sync_copy(idx_hbm.at[jnp.int32(0), pl.ds(base, per_w)], idx0)
        pltpu.async_copy(table_hbm.at[idx0], val0, g0)

        @pl.loop(jnp.int32(0), jnp.int32(n_it), step=2)
        def body(it0):
            for b in range(2):
                it = it0 + jnp.int32(b)
                idx_c, val_c, g_c, o_c = slots[b]
                idx_n, val_n, g_n, o_n = slots[1 - b]

                @pl.when(it + 1 < n_it)
                def _():
                    pltpu.sync_copy(
                        idx_hbm.at[it + 1, pl.ds(base, per_w)], idx_n
                    )
                    @pl.when(it >= 1)
                    def _():
                        pltpu.make_async_copy(
                            val_n, out_hbm.at[it - 1, pl.ds(base, per_w)], o_n
                        ).wait()

                    pltpu.async_copy(table_hbm.at[idx_n], val_n, g_n)

                pltpu.make_async_copy(table_hbm.at[idx_c], val_c, g_c).wait()
                pltpu.async_copy(
                    val_c, out_hbm.at[it, pl.ds(base, per_w)], o_c
                )

        pltpu.make_async_copy(
            val0, out_hbm.at[jnp.int32(n_it - 2), pl.ds(base, per_w)], o0
        ).wait()
        pltpu.make_async_copy(
            val1, out_hbm.at[jnp.int32(n_it - 1), pl.ds(base, per_w)], o1
        ).wait()

    return k(table_1d, idx)


def _mlp_body(x_ref, w1_ref, b1_ref, w2_ref, b2_ref, w3_ref, b3_ref, o_ref):
    x = x_ref[...]
    h = jnp.dot(w1_ref[...], x, preferred_element_type=jnp.float32) + b1_ref[...]
    h = jnp.maximum(h, 0.0)
    h = jnp.dot(w2_ref[...], h, preferred_element_type=jnp.float32) + b2_ref[...]
    h = jnp.maximum(h, 0.0)
    o = jnp.dot(w3_ref[...], h, preferred_element_type=jnp.float32) + b3_ref[...]
    sig = 1.0 / (1.0 + jnp.exp(-o))
    sp = jnp.maximum(o, 0.0) + jnp.log(1.0 + jnp.exp(-jnp.abs(o)))
    row = lax.broadcasted_iota(jnp.int32, o.shape, 0)
    o_ref[...] = jnp.where(row < 3, sig, sp)


def _mlp(feats, W1t, b1, W2t, b2, W3t, b3):
    blk = 2048
    grid = BATCH // blk
    full = lambda a: pl.BlockSpec(a.shape, lambda i: tuple(i * 0 for _ in a.shape))
    return pl.pallas_call(
        _mlp_body,
        grid=(grid,),
        in_specs=[
            pl.BlockSpec((MLP_IN, blk), lambda i: (i * 0, i)),
            full(W1t), full(b1), full(W2t), full(b2), full(W3t), full(b3),
        ],
        out_specs=pl.BlockSpec((4, blk), lambda i: (i * 0, i)),
        out_shape=jax.ShapeDtypeStruct((4, BATCH), jnp.float32),
    )(feats, W1t, b1, W2t, b2, W3t, b3)


def kernel(positions, tables, W1, b1, W2, b2, W3, b3):
    f32 = jnp.float32
    positions = positions.astype(f32)
    tables = tables.astype(f32)
    W1, b1, W2, b2, W3, b3 = (a.astype(f32) for a in (W1, b1, W2, b2, W3, b3))

    idx = _compute_indices(positions.T)
    table_1d = (
        tables.reshape(NUM_LEVELS, TABLE_SIZE // 128, 128, F_PER_LEVEL)
        .transpose(0, 1, 3, 2)
        .reshape(-1)
    )
    feats = _gather_sc(table_1d, idx)
    y = _mlp(
        feats,
        W1.T,
        b1.reshape(HIDDEN, 1),
        W2.T,
        b2.reshape(HIDDEN, 1),
        W3.T,
        b3.reshape(4, 1),
    )
    yt = y.T.astype(jnp.float64)
    return yt[:, :3], yt[:, 3:4]


# speedup vs baseline: 64.7768x; 1.0094x over previous
import functools

import jax
import jax.numpy as jnp
from jax import lax
from jax.experimental import pallas as pl
from jax.experimental.pallas import tpu as pltpu
from jax.experimental.pallas import tpu_sc as plsc

NUM_LEVELS = 16
F_PER_LEVEL = 2
LOG2_HASHMAP = 19
TABLE_SIZE = 2 ** LOG2_HASHMAP
BATCH = 262144
HIDDEN = 64
MLP_IN = NUM_LEVELS * F_PER_LEVEL

_RES = [int(16 * (2048 / 16) ** (l / (NUM_LEVELS - 1))) for l in range(NUM_LEVELS)]
_M1 = 2654435761 - (1 << 32)
_M2 = 805459861

_NW = 32
_NSLICE = 2


def _hash_body(pos_ref, res_ref, idx_ref):
    p = pos_ref[...]
    res = res_ref[...]
    r = lax.broadcasted_iota(jnp.int32, (32, 1), 0)
    lvl = r >> 1
    feat = r & 1
    c0 = (p[0:1, :] * res).astype(jnp.int32)
    c1 = (p[1:2, :] * res).astype(jnp.int32)
    c2 = (p[2:3, :] * res).astype(jnp.int32)
    h = c0 ^ (c1 * jnp.int32(_M1)) ^ (c2 * jnp.int32(_M2))
    e = h & jnp.int32(TABLE_SIZE - 1)
    idx_ref[...] = (
        (lvl << 20)
        + ((e >> 7) << 8)
        + (feat << 7)
        + (e & jnp.int32(127))
    )


def _compute_indices(pos_t, n):
    blk = 2048
    grid = n // blk
    res = jnp.repeat(jnp.array(_RES, dtype=jnp.float32), 2).reshape(32, 1)
    return pl.pallas_call(
        _hash_body,
        grid=(grid,),
        in_specs=[
            pl.BlockSpec((3, blk), lambda i: (i * 0, i)),
            pl.BlockSpec((32, 1), lambda i: (i * 0, i * 0)),
        ],
        out_specs=pl.BlockSpec((32, blk), lambda i: (i * 0, i)),
        out_shape=jax.ShapeDtypeStruct((MLP_IN, n), jnp.int32),
    )(pos_t, res)


def _gather_sc(table_1d, idx, n):
    per_w = n // _NW
    n_it = MLP_IN
    mesh = plsc.VectorSubcoreMesh(core_axis_name="c", subcore_axis_name="s")

    @functools.partial(
        pl.kernel,
        out_type=jax.ShapeDtypeStruct((MLP_IN, n), jnp.float32),
        mesh=mesh,
        compiler_params=pltpu.CompilerParams(use_tc_tiling_on_sc=False),
        scratch_types=[
            pltpu.VMEM((per_w,), jnp.int32),
            pltpu.VMEM((per_w,), jnp.int32),
            pltpu.VMEM((per_w,), jnp.float32),
            pltpu.VMEM((per_w,), jnp.float32),
            pltpu.SemaphoreType.DMA,
            pltpu.SemaphoreType.DMA,
            pltpu.SemaphoreType.DMA,
            pltpu.SemaphoreType.DMA,
        ],
    )
    def k(table_hbm, idx_hbm, out_hbm, idx0, idx1, val0, val1, g0, g1, o0, o1):
        wid = (lax.axis_index("s") * 2 + lax.axis_index("c")).astype(jnp.int32)
        base = wid * jnp.int32(per_w)
        slots = ((idx0, val0, g0, o0), (idx1, val1, g1, o1))

        pltpu.sync_copy(idx_hbm.at[jnp.int32(0), pl.ds(base, per_w)], idx0)
        pltpu.async_copy(table_hbm.at[idx0], val0, g0)

        @pl.loop(jnp.int32(0), jnp.int32(n_it), step=2)
        def body(it0):
            for b in range(2):
                it = it0 + jnp.int32(b)
                idx_c, val_c, g_c, o_c = slots[b]
                idx_n, val_n, g_n, o_n = slots[1 - b]

                @pl.when(it + 1 < n_it)
                def _():
                    pltpu.sync_copy(
                        idx_hbm.at[it + 1, pl.ds(base, per_w)], idx_n
                    )
                    @pl.when(it >= 1)
                    def _():
                        pltpu.make_async_copy(
                            val_n, out_hbm.at[it - 1, pl.ds(base, per_w)], o_n
                        ).wait()

                    pltpu.async_copy(table_hbm.at[idx_n], val_n, g_n)

                pltpu.make_async_copy(table_hbm.at[idx_c], val_c, g_c).wait()
                pltpu.async_copy(
                    val_c, out_hbm.at[it, pl.ds(base, per_w)], o_c
                )

        pltpu.make_async_copy(
            val0, out_hbm.at[jnp.int32(n_it - 2), pl.ds(base, per_w)], o0
        ).wait()
        pltpu.make_async_copy(
            val1, out_hbm.at[jnp.int32(n_it - 1), pl.ds(base, per_w)], o1
        ).wait()

    return k(table_1d, idx)


def _mlp_body(x_ref, w1_ref, b1_ref, w2_ref, b2_ref, w3_ref, b3_ref, o_ref):
    x = x_ref[...]
    h = jnp.dot(w1_ref[...], x, preferred_element_type=jnp.float32) + b1_ref[...]
    h = jnp.maximum(h, 0.0)
    h = jnp.dot(w2_ref[...], h, preferred_element_type=jnp.float32) + b2_ref[...]
    h = jnp.maximum(h, 0.0)
    o = jnp.dot(w3_ref[...], h, preferred_element_type=jnp.float32) + b3_ref[...]
    sig = 1.0 / (1.0 + jnp.exp(-o))
    sp = jnp.maximum(o, 0.0) + jnp.log(1.0 + jnp.exp(-jnp.abs(o)))
    row = lax.broadcasted_iota(jnp.int32, o.shape, 0)
    o_ref[...] = jnp.where(row < 3, sig, sp)


def _mlp(feats, W1t, b1, W2t, b2, W3t, b3, n):
    blk = 2048
    grid = n // blk
    full = lambda a: pl.BlockSpec(a.shape, lambda i: tuple(i * 0 for _ in a.shape))
    return pl.pallas_call(
        _mlp_body,
        grid=(grid,),
        in_specs=[
            pl.BlockSpec((MLP_IN, blk), lambda i: (i * 0, i)),
            full(W1t), full(b1), full(W2t), full(b2), full(W3t), full(b3),
        ],
        out_specs=pl.BlockSpec((4, blk), lambda i: (i * 0, i)),
        out_shape=jax.ShapeDtypeStruct((4, n), jnp.float32),
    )(feats, W1t, b1, W2t, b2, W3t, b3)


def kernel(positions, tables, W1, b1, W2, b2, W3, b3):
    f32 = jnp.float32
    positions = positions.astype(f32)
    tables = tables.astype(f32)
    W1, b1, W2, b2, W3, b3 = (a.astype(f32) for a in (W1, b1, W2, b2, W3, b3))

    table_1d = (
        tables.reshape(NUM_LEVELS, TABLE_SIZE // 128, 128, F_PER_LEVEL)
        .transpose(0, 1, 3, 2)
        .reshape(-1)
    )
    pos_t = positions.T
    w_args = (W1.T, b1.reshape(HIDDEN, 1), W2.T, b2.reshape(HIDDEN, 1),
              W3.T, b3.reshape(4, 1))
    ns = _NSLICE
    bs = BATCH // ns
    ys = []
    for s in range(ns):
        sl = slice(s * bs, (s + 1) * bs)
        idx_s = _compute_indices(pos_t[:, sl], bs)
        feats_s = _gather_sc(table_1d, idx_s, bs)
        ys.append(_mlp(feats_s, *w_args, bs))
    y = jnp.concatenate(ys, axis=1) if ns > 1 else ys[0]
    yt = y.T.astype(jnp.float64)
    return yt[:, :3], yt[:, 3:4]


# speedup vs baseline: 71.4026x; 1.1023x over previous
import functools

import jax
import jax.numpy as jnp
from jax import lax
from jax.experimental import pallas as pl
from jax.experimental.pallas import tpu as pltpu
from jax.experimental.pallas import tpu_sc as plsc

NUM_LEVELS = 16
F_PER_LEVEL = 2
LOG2_HASHMAP = 19
TABLE_SIZE = 2 ** LOG2_HASHMAP
BATCH = 262144
HIDDEN = 64
MLP_IN = NUM_LEVELS * F_PER_LEVEL

_RES = [int(16 * (2048 / 16) ** (l / (NUM_LEVELS - 1))) for l in range(NUM_LEVELS)]
_M1 = 2654435761 - (1 << 32)
_M2 = 805459861

_NW = 32
_NSLICE = 4


def _hash_body(pos_ref, res_ref, idx_ref):
    p = pos_ref[...]
    res = res_ref[...]
    r = lax.broadcasted_iota(jnp.int32, (32, 1), 0)
    lvl = r >> 1
    feat = r & 1
    c0 = (p[0:1, :] * res).astype(jnp.int32)
    c1 = (p[1:2, :] * res).astype(jnp.int32)
    c2 = (p[2:3, :] * res).astype(jnp.int32)
    h = c0 ^ (c1 * jnp.int32(_M1)) ^ (c2 * jnp.int32(_M2))
    e = h & jnp.int32(TABLE_SIZE - 1)
    idx_ref[...] = (
        (lvl << 20)
        + ((e >> 7) << 8)
        + (feat << 7)
        + (e & jnp.int32(127))
    )


def _compute_indices(pos_t, n):
    blk = 2048
    grid = n // blk
    res = jnp.repeat(jnp.array(_RES, dtype=jnp.float32), 2).reshape(32, 1)
    return pl.pallas_call(
        _hash_body,
        grid=(grid,),
        in_specs=[
            pl.BlockSpec((3, blk), lambda i: (i * 0, i)),
            pl.BlockSpec((32, 1), lambda i: (i * 0, i * 0)),
        ],
        out_specs=pl.BlockSpec((32, blk), lambda i: (i * 0, i)),
        out_shape=jax.ShapeDtypeStruct((MLP_IN, n), jnp.int32),
    )(pos_t, res)


def _gather_sc(table_1d, idx, n):
    per_w = n // _NW
    n_it = MLP_IN
    mesh = plsc.VectorSubcoreMesh(core_axis_name="c", subcore_axis_name="s")

    @functools.partial(
        pl.kernel,
        out_type=jax.ShapeDtypeStruct((MLP_IN, n), jnp.float32),
        mesh=mesh,
        compiler_params=pltpu.CompilerParams(use_tc_tiling_on_sc=False),
        scratch_types=[
            pltpu.VMEM((per_w,), jnp.int32),
            pltpu.VMEM((per_w,), jnp.int32),
            pltpu.VMEM((per_w,), jnp.float32),
            pltpu.VMEM((per_w,), jnp.float32),
            pltpu.SemaphoreType.DMA,
            pltpu.SemaphoreType.DMA,
            pltpu.SemaphoreType.DMA,
            pltpu.SemaphoreType.DMA,
        ],
    )
    def k(table_hbm, idx_hbm, out_hbm, idx0, idx1, val0, val1, g0, g1, o0, o1):
        wid = (lax.axis_index("s") * 2 + lax.axis_index("c")).astype(jnp.int32)
        base = wid * jnp.int32(per_w)
        slots = ((idx0, val0, g0, o0), (idx1, val1, g1, o1))

        pltpu.sync_copy(idx_hbm.at[jnp.int32(0), pl.ds(base, per_w)], idx0)
        pltpu.async_copy(table_hbm.at[idx0], val0, g0)

        @pl.loop(jnp.int32(0), jnp.int32(n_it), step=2)
        def body(it0):
            for b in range(2):
                it = it0 + jnp.int32(b)
                idx_c, val_c, g_c, o_c = slots[b]
                idx_n, val_n, g_n, o_n = slots[1 - b]

                @pl.when(it + 1 < n_it)
                def _():
                    pltpu.sync_copy(
                        idx_hbm.at[it + 1, pl.ds(base, per_w)], idx_n
                    )
                    @pl.when(it >= 1)
                    def _():
                        pltpu.make_async_copy(
                            val_n, out_hbm.at[it - 1, pl.ds(base, per_w)], o_n
                        ).wait()

                    pltpu.async_copy(table_hbm.at[idx_n], val_n, g_n)

                pltpu.make_async_copy(table_hbm.at[idx_c], val_c, g_c).wait()
                pltpu.async_copy(
                    val_c, out_hbm.at[it, pl.ds(base, per_w)], o_c
                )

        pltpu.make_async_copy(
            val0, out_hbm.at[jnp.int32(n_it - 2), pl.ds(base, per_w)], o0
        ).wait()
        pltpu.make_async_copy(
            val1, out_hbm.at[jnp.int32(n_it - 1), pl.ds(base, per_w)], o1
        ).wait()

    return k(table_1d, idx)


def _mlp_body(x_ref, w1_ref, b1_ref, w2_ref, b2_ref, w3_ref, b3_ref, o_ref):
    x = x_ref[...]
    h = jnp.dot(w1_ref[...], x, preferred_element_type=jnp.float32) + b1_ref[...]
    h = jnp.maximum(h, 0.0)
    h = jnp.dot(w2_ref[...], h, preferred_element_type=jnp.float32) + b2_ref[...]
    h = jnp.maximum(h, 0.0)
    o = jnp.dot(w3_ref[...], h, preferred_element_type=jnp.float32) + b3_ref[...]
    sig = 1.0 / (1.0 + jnp.exp(-o))
    sp = jnp.maximum(o, 0.0) + jnp.log(1.0 + jnp.exp(-jnp.abs(o)))
    row = lax.broadcasted_iota(jnp.int32, o.shape, 0)
    o_ref[...] = jnp.where(row < 3, sig, sp)


def _mlp(feats, W1t, b1, W2t, b2, W3t, b3, n):
    blk = 2048
    grid = n // blk
    full = lambda a: pl.BlockSpec(a.shape, lambda i: tuple(i * 0 for _ in a.shape))
    return pl.pallas_call(
        _mlp_body,
        grid=(grid,),
        in_specs=[
            pl.BlockSpec((MLP_IN, blk), lambda i: (i * 0, i)),
            full(W1t), full(b1), full(W2t), full(b2), full(W3t), full(b3),
        ],
        out_specs=pl.BlockSpec((4, blk), lambda i: (i * 0, i)),
        out_shape=jax.ShapeDtypeStruct((4, n), jnp.float32),
    )(feats, W1t, b1, W2t, b2, W3t, b3)


def kernel(positions, tables, W1, b1, W2, b2, W3, b3):
    f32 = jnp.float32
    positions = positions.astype(f32)
    tables = tables.astype(f32)
    W1, b1, W2, b2, W3, b3 = (a.astype(f32) for a in (W1, b1, W2, b2, W3, b3))

    table_1d = (
        tables.reshape(NUM_LEVELS, TABLE_SIZE // 128, 128, F_PER_LEVEL)
        .transpose(0, 1, 3, 2)
        .reshape(-1)
    )
    pos_t = positions.T
    w_args = (W1.T, b1.reshape(HIDDEN, 1), W2.T, b2.reshape(HIDDEN, 1),
              W3.T, b3.reshape(4, 1))
    ns = _NSLICE
    bs = BATCH // ns
    ys = []
    for s in range(ns):
        sl = slice(s * bs, (s + 1) * bs)
        idx_s = _compute_indices(pos_t[:, sl], bs)
        feats_s = _gather_sc(table_1d, idx_s, bs)
        ys.append(_mlp(feats_s, *w_args, bs))
    y = jnp.concatenate(ys, axis=1) if ns > 1 else ys[0]
    yt = y.T.astype(jnp.float64)
    return yt[:, :3], yt[:, 3:4]


# speedup vs baseline: 73.1466x; 1.0244x over previous
import functools

import jax
import jax.numpy as jnp
from jax import lax
from jax.experimental import pallas as pl
from jax.experimental.pallas import tpu as pltpu
from jax.experimental.pallas import tpu_sc as plsc

NUM_LEVELS = 16
F_PER_LEVEL = 2
LOG2_HASHMAP = 19
TABLE_SIZE = 2 ** LOG2_HASHMAP
BATCH = 262144
HIDDEN = 64
MLP_IN = NUM_LEVELS * F_PER_LEVEL

_RES = [int(16 * (2048 / 16) ** (l / (NUM_LEVELS - 1))) for l in range(NUM_LEVELS)]
_M1 = 2654435761 - (1 << 32)
_M2 = 805459861

_NW = 32
_NSLICE = 4


def _hash_body(pos_ref, res_ref, idx_ref):
    p = pos_ref[...]
    res = res_ref[...]
    r = lax.broadcasted_iota(jnp.int32, (32, 1), 0)
    lvl = r >> 1
    feat = r & 1
    c0 = (p[0:1, :] * res).astype(jnp.int32)
    c1 = (p[1:2, :] * res).astype(jnp.int32)
    c2 = (p[2:3, :] * res).astype(jnp.int32)
    h = c0 ^ (c1 * jnp.int32(_M1)) ^ (c2 * jnp.int32(_M2))
    e = h & jnp.int32(TABLE_SIZE - 1)
    idx_ref[...] = (
        (lvl << 20)
        + ((e >> 7) << 8)
        + (feat << 7)
        + (e & jnp.int32(127))
    )


def _compute_indices(pos_t, n):
    blk = 2048
    grid = n // blk
    res = jnp.repeat(jnp.array(_RES, dtype=jnp.float32), 2).reshape(32, 1)
    return pl.pallas_call(
        _hash_body,
        grid=(grid,),
        in_specs=[
            pl.BlockSpec((3, blk), lambda i: (i * 0, i)),
            pl.BlockSpec((32, 1), lambda i: (i * 0, i * 0)),
        ],
        out_specs=pl.BlockSpec((32, blk), lambda i: (i * 0, i)),
        out_shape=jax.ShapeDtypeStruct((MLP_IN, n), jnp.int32),
    )(pos_t, res)


def _gather_sc(table_1d, idx, n):
    per_w = n // _NW
    n_it = MLP_IN
    mesh = plsc.VectorSubcoreMesh(core_axis_name="c", subcore_axis_name="s")

    @functools.partial(
        pl.kernel,
        out_type=jax.ShapeDtypeStruct((MLP_IN, n), jnp.float32),
        mesh=mesh,
        compiler_params=pltpu.CompilerParams(use_tc_tiling_on_sc=False),
        scratch_types=[
            pltpu.VMEM((per_w,), jnp.int32),
            pltpu.VMEM((per_w,), jnp.int32),
            pltpu.VMEM((per_w,), jnp.float32),
            pltpu.VMEM((per_w,), jnp.float32),
            pltpu.SemaphoreType.DMA,
            pltpu.SemaphoreType.DMA,
            pltpu.SemaphoreType.DMA,
            pltpu.SemaphoreType.DMA,
        ],
    )
    def k(table_hbm, idx_hbm, out_hbm, idx0, idx1, val0, val1, g0, g1, o0, o1):
        wid = (lax.axis_index("s") * 2 + lax.axis_index("c")).astype(jnp.int32)
        base = wid * jnp.int32(per_w)
        slots = ((idx0, val0, g0, o0), (idx1, val1, g1, o1))

        pltpu.sync_copy(idx_hbm.at[jnp.int32(0), pl.ds(base, per_w)], idx0)
        pltpu.async_copy(table_hbm.at[idx0], val0, g0)

        @pl.loop(jnp.int32(0), jnp.int32(n_it), step=2)
        def body(it0):
            for b in range(2):
                it = it0 + jnp.int32(b)
                idx_c, val_c, g_c, o_c = slots[b]
                idx_n, val_n, g_n, o_n = slots[1 - b]

                @pl.when(it + 1 < n_it)
                def _():
                    pltpu.sync_copy(
                        idx_hbm.at[it + 1, pl.ds(base, per_w)], idx_n
                    )
                    @pl.when(it >= 1)
                    def _():
                        pltpu.make_async_copy(
                            val_n, out_hbm.at[it - 1, pl.ds(base, per_w)], o_n
                        ).wait()

                    pltpu.async_copy(table_hbm.at[idx_n], val_n, g_n)

                pltpu.make_async_copy(table_hbm.at[idx_c], val_c, g_c).wait()
                pltpu.async_copy(
                    val_c, out_hbm.at[it, pl.ds(base, per_w)], o_c
                )

        pltpu.make_async_copy(
            val0, out_hbm.at[jnp.int32(n_it - 2), pl.ds(base, per_w)], o0
        ).wait()
        pltpu.make_async_copy(
            val1, out_hbm.at[jnp.int32(n_it - 1), pl.ds(base, per_w)], o1
        ).wait()

    return k(table_1d, idx)


def _mlp_body(x_ref, w1_ref, b1_ref, w2_ref, b2_ref, w3_ref, b3_ref, o_ref):
    x = x_ref[...]
    h = jnp.dot(w1_ref[...], x, preferred_element_type=jnp.float32) + b1_ref[...]
    h = jnp.maximum(h, 0.0)
    h = jnp.dot(w2_ref[...], h, preferred_element_type=jnp.float32) + b2_ref[...]
    h = jnp.maximum(h, 0.0)
    o = jnp.dot(w3_ref[...], h, preferred_element_type=jnp.float32) + b3_ref[...]
    sig = 1.0 / (1.0 + jnp.exp(-o))
    sp = jnp.maximum(o, 0.0) + jnp.log(1.0 + jnp.exp(-jnp.abs(o)))
    row = lax.broadcasted_iota(jnp.int32, o.shape, 0)
    o_ref[...] = jnp.where(row < 3, sig, sp)


def _mlp(feats, W1t, b1, W2t, b2, W3t, b3, n):
    blk = 8192
    grid = n // blk
    full = lambda a: pl.BlockSpec(a.shape, lambda i: tuple(i * 0 for _ in a.shape))
    return pl.pallas_call(
        _mlp_body,
        grid=(grid,),
        in_specs=[
            pl.BlockSpec((MLP_IN, blk), lambda i: (i * 0, i)),
            full(W1t), full(b1), full(W2t), full(b2), full(W3t), full(b3),
        ],
        out_specs=pl.BlockSpec((4, blk), lambda i: (i * 0, i)),
        out_shape=jax.ShapeDtypeStruct((4, n), jnp.float32),
    )(feats, W1t, b1, W2t, b2, W3t, b3)


def kernel(positions, tables, W1, b1, W2, b2, W3, b3):
    f32 = jnp.float32
    positions = positions.astype(f32)
    tables = tables.astype(f32)
    W1, b1, W2, b2, W3, b3 = (a.astype(f32) for a in (W1, b1, W2, b2, W3, b3))

    table_1d = (
        tables.reshape(NUM_LEVELS, TABLE_SIZE // 128, 128, F_PER_LEVEL)
        .transpose(0, 1, 3, 2)
        .reshape(-1)
    )
    pos_t = positions.T
    w_args = (W1.T, b1.reshape(HIDDEN, 1), W2.T, b2.reshape(HIDDEN, 1),
              W3.T, b3.reshape(4, 1))
    ns = _NSLICE
    bs = BATCH // ns
    ys = []
    for s in range(ns):
        sl = slice(s * bs, (s + 1) * bs)
        idx_s = _compute_indices(pos_t[:, sl], bs)
        feats_s = _gather_sc(table_1d, idx_s, bs)
        ys.append(_mlp(feats_s, *w_args, bs))
    y = jnp.concatenate(ys, axis=1) if ns > 1 else ys[0]
    yt = y.T.astype(jnp.float64)
    return yt[:, :3], yt[:, 3:4]


# speedup vs baseline: 78.5497x; 1.0739x over previous
import functools

import jax
import jax.numpy as jnp
from jax import lax
from jax.experimental import pallas as pl
from jax.experimental.pallas import tpu as pltpu
from jax.experimental.pallas import tpu_sc as plsc

NUM_LEVELS = 16
F_PER_LEVEL = 2
LOG2_HASHMAP = 19
TABLE_SIZE = 2 ** LOG2_HASHMAP
BATCH = 262144
HIDDEN = 64
MLP_IN = NUM_LEVELS * F_PER_LEVEL

_RES = [int(16 * (2048 / 16) ** (l / (NUM_LEVELS - 1))) for l in range(NUM_LEVELS)]
_M1 = 2654435761 - (1 << 32)
_M2 = 805459861

_NW = 32
_NSLICE = 4


def _hash_body(pos_ref, res_ref, idx_ref):
    p = pos_ref[...]
    res = res_ref[...]
    r = lax.broadcasted_iota(jnp.int32, (32, 1), 0)
    lvl = r >> 1
    feat = r & 1
    c0 = (p[0:1, :] * res).astype(jnp.int32)
    c1 = (p[1:2, :] * res).astype(jnp.int32)
    c2 = (p[2:3, :] * res).astype(jnp.int32)
    h = c0 ^ (c1 * jnp.int32(_M1)) ^ (c2 * jnp.int32(_M2))
    e = h & jnp.int32(TABLE_SIZE - 1)
    idx_ref[...] = (
        (lvl << 20)
        + ((e >> 7) << 8)
        + (feat << 7)
        + (e & jnp.int32(127))
    )


def _compute_indices(pos_t, n):
    blk = 8192
    grid = n // blk
    res = jnp.repeat(jnp.array(_RES, dtype=jnp.float32), 2).reshape(32, 1)
    return pl.pallas_call(
        _hash_body,
        grid=(grid,),
        in_specs=[
            pl.BlockSpec((3, blk), lambda i: (i * 0, i)),
            pl.BlockSpec((32, 1), lambda i: (i * 0, i * 0)),
        ],
        out_specs=pl.BlockSpec((32, blk), lambda i: (i * 0, i)),
        out_shape=jax.ShapeDtypeStruct((MLP_IN, n), jnp.int32),
    )(pos_t, res)


def _gather_sc(table_1d, idx, n):
    per_w = n // _NW
    n_it = MLP_IN
    mesh = plsc.VectorSubcoreMesh(core_axis_name="c", subcore_axis_name="s")

    @functools.partial(
        pl.kernel,
        out_type=jax.ShapeDtypeStruct((MLP_IN, n), jnp.float32),
        mesh=mesh,
        compiler_params=pltpu.CompilerParams(use_tc_tiling_on_sc=False),
        scratch_types=[
            pltpu.VMEM((per_w,), jnp.int32),
            pltpu.VMEM((per_w,), jnp.int32),
            pltpu.VMEM((per_w,), jnp.float32),
            pltpu.VMEM((per_w,), jnp.float32),
            pltpu.SemaphoreType.DMA,
            pltpu.SemaphoreType.DMA,
            pltpu.SemaphoreType.DMA,
            pltpu.SemaphoreType.DMA,
        ],
    )
    def k(table_hbm, idx_hbm, out_hbm, idx0, idx1, val0, val1, g0, g1, o0, o1):
        wid = (lax.axis_index("s") * 2 + lax.axis_index("c")).astype(jnp.int32)
        base = wid * jnp.int32(per_w)
        slots = ((idx0, val0, g0, o0), (idx1, val1, g1, o1))

        pltpu.sync_copy(idx_hbm.at[jnp.int32(0), pl.ds(base, per_w)], idx0)
        pltpu.async_copy(table_hbm.at[idx0], val0, g0)

        @pl.loop(jnp.int32(0), jnp.int32(n_it), step=2)
        def body(it0):
            for b in range(2):
                it = it0 + jnp.int32(b)
                idx_c, val_c, g_c, o_c = slots[b]
                idx_n, val_n, g_n, o_n = slots[1 - b]

                @pl.when(it + 1 < n_it)
                def _():
                    pltpu.sync_copy(
                        idx_hbm.at[it + 1, pl.ds(base, per_w)], idx_n
                    )
                    @pl.when(it >= 1)
                    def _():
                        pltpu.make_async_copy(
                            val_n, out_hbm.at[it - 1, pl.ds(base, per_w)], o_n
                        ).wait()

                    pltpu.async_copy(table_hbm.at[idx_n], val_n, g_n)

                pltpu.make_async_copy(table_hbm.at[idx_c], val_c, g_c).wait()
                pltpu.async_copy(
                    val_c, out_hbm.at[it, pl.ds(base, per_w)], o_c
                )

        pltpu.make_async_copy(
            val0, out_hbm.at[jnp.int32(n_it - 2), pl.ds(base, per_w)], o0
        ).wait()
        pltpu.make_async_copy(
            val1, out_hbm.at[jnp.int32(n_it - 1), pl.ds(base, per_w)], o1
        ).wait()

    return k(table_1d, idx)


def _mlp_body(x_ref, w1_ref, b1_ref, w2_ref, b2_ref, w3_ref, b3_ref, o_ref):
    x = x_ref[...]
    h = jnp.dot(w1_ref[...], x, preferred_element_type=jnp.float32) + b1_ref[...]
    h = jnp.maximum(h, 0.0)
    h = jnp.dot(w2_ref[...], h, preferred_element_type=jnp.float32) + b2_ref[...]
    h = jnp.maximum(h, 0.0)
    o = jnp.dot(w3_ref[...], h, preferred_element_type=jnp.float32) + b3_ref[...]
    sig = 1.0 / (1.0 + jnp.exp(-o))
    sp = jnp.maximum(o, 0.0) + jnp.log(1.0 + jnp.exp(-jnp.abs(o)))
    row = lax.broadcasted_iota(jnp.int32, o.shape, 0)
    o_ref[...] = jnp.where(row < 3, sig, sp)


def _mlp(feats, W1t, b1, W2t, b2, W3t, b3, n):
    blk = 8192
    grid = n // blk
    full = lambda a: pl.BlockSpec(a.shape, lambda i: tuple(i * 0 for _ in a.shape))
    return pl.pallas_call(
        _mlp_body,
        grid=(grid,),
        in_specs=[
            pl.BlockSpec((MLP_IN, blk), lambda i: (i * 0, i)),
            full(W1t), full(b1), full(W2t), full(b2), full(W3t), full(b3),
        ],
        out_specs=pl.BlockSpec((4, blk), lambda i: (i * 0, i)),
        out_shape=jax.ShapeDtypeStruct((4, n), jnp.float32),
    )(feats, W1t, b1, W2t, b2, W3t, b3)


def kernel(positions, tables, W1, b1, W2, b2, W3, b3):
    f32 = jnp.float32
    positions = positions.astype(f32)
    tables = tables.astype(f32)
    W1, b1, W2, b2, W3, b3 = (a.astype(f32) for a in (W1, b1, W2, b2, W3, b3))

    table_1d = (
        tables.reshape(NUM_LEVELS, TABLE_SIZE // 128, 128, F_PER_LEVEL)
        .transpose(0, 1, 3, 2)
        .reshape(-1)
    )
    pos_t = positions.T
    w_args = (W1.T, b1.reshape(HIDDEN, 1), W2.T, b2.reshape(HIDDEN, 1),
              W3.T, b3.reshape(4, 1))
    ns = _NSLICE
    bs = BATCH // ns
    ys = []
    for s in range(ns):
        sl = slice(s * bs, (s + 1) * bs)
        idx_s = _compute_indices(pos_t[:, sl], bs)
        feats_s = _gather_sc(table_1d, idx_s, bs)
        ys.append(_mlp(feats_s, *w_args, bs))
    y = jnp.concatenate(ys, axis=1) if ns > 1 else ys[0]
    yt = y.T.astype(jnp.float64)
    return yt[:, :3], yt[:, 3:4]


# speedup vs baseline: 78.7433x; 1.0025x over previous
import functools

import jax
import jax.numpy as jnp
from jax import lax
from jax.experimental import pallas as pl
from jax.experimental.pallas import tpu as pltpu
from jax.experimental.pallas import tpu_sc as plsc

NUM_LEVELS = 16
F_PER_LEVEL = 2
LOG2_HASHMAP = 19
TABLE_SIZE = 2 ** LOG2_HASHMAP
BATCH = 262144
HIDDEN = 64
MLP_IN = NUM_LEVELS * F_PER_LEVEL

_RES = [int(16 * (2048 / 16) ** (l / (NUM_LEVELS - 1))) for l in range(NUM_LEVELS)]
_M1 = 2654435761 - (1 << 32)
_M2 = 805459861

_NW = 32
_NSLICE = 8


def _hash_body(pos_ref, res_ref, idx_ref):
    p = pos_ref[...]
    res = res_ref[...]
    r = lax.broadcasted_iota(jnp.int32, (32, 1), 0)
    lvl = r >> 1
    feat = r & 1
    c0 = (p[0:1, :] * res).astype(jnp.int32)
    c1 = (p[1:2, :] * res).astype(jnp.int32)
    c2 = (p[2:3, :] * res).astype(jnp.int32)
    h = c0 ^ (c1 * jnp.int32(_M1)) ^ (c2 * jnp.int32(_M2))
    e = h & jnp.int32(TABLE_SIZE - 1)
    idx_ref[...] = (
        (lvl << 20)
        + ((e >> 7) << 8)
        + (feat << 7)
        + (e & jnp.int32(127))
    )


def _compute_indices(pos_t, n):
    blk = 8192
    grid = n // blk
    res = jnp.repeat(jnp.array(_RES, dtype=jnp.float32), 2).reshape(32, 1)
    return pl.pallas_call(
        _hash_body,
        grid=(grid,),
        in_specs=[
            pl.BlockSpec((3, blk), lambda i: (i * 0, i)),
            pl.BlockSpec((32, 1), lambda i: (i * 0, i * 0)),
        ],
        out_specs=pl.BlockSpec((32, blk), lambda i: (i * 0, i)),
        out_shape=jax.ShapeDtypeStruct((MLP_IN, n), jnp.int32),
    )(pos_t, res)


def _gather_sc(table_1d, idx, n):
    per_w = n // _NW
    n_it = MLP_IN
    mesh = plsc.VectorSubcoreMesh(core_axis_name="c", subcore_axis_name="s")

    @functools.partial(
        pl.kernel,
        out_type=jax.ShapeDtypeStruct((MLP_IN, n), jnp.float32),
        mesh=mesh,
        compiler_params=pltpu.CompilerParams(use_tc_tiling_on_sc=False),
        scratch_types=[
            pltpu.VMEM((per_w,), jnp.int32),
            pltpu.VMEM((per_w,), jnp.int32),
            pltpu.VMEM((per_w,), jnp.float32),
            pltpu.VMEM((per_w,), jnp.float32),
            pltpu.SemaphoreType.DMA,
            pltpu.SemaphoreType.DMA,
            pltpu.SemaphoreType.DMA,
            pltpu.SemaphoreType.DMA,
        ],
    )
    def k(table_hbm, idx_hbm, out_hbm, idx0, idx1, val0, val1, g0, g1, o0, o1):
        wid = (lax.axis_index("s") * 2 + lax.axis_index("c")).astype(jnp.int32)
        base = wid * jnp.int32(per_w)
        slots = ((idx0, val0, g0, o0), (idx1, val1, g1, o1))

        pltpu.sync_copy(idx_hbm.at[jnp.int32(0), pl.ds(base, per_w)], idx0)
        pltpu.async_copy(table_hbm.at[idx0], val0, g0)

        @pl.loop(jnp.int32(0), jnp.int32(n_it), step=2)
        def body(it0):
            for b in range(2):
                it = it0 + jnp.int32(b)
                idx_c, val_c, g_c, o_c = slots[b]
                idx_n, val_n, g_n, o_n = slots[1 - b]

                @pl.when(it + 1 < n_it)
                def _():
                    pltpu.sync_copy(
                        idx_hbm.at[it + 1, pl.ds(base, per_w)], idx_n
                    )
                    @pl.when(it >= 1)
                    def _():
                        pltpu.make_async_copy(
                            val_n, out_hbm.at[it - 1, pl.ds(base, per_w)], o_n
                        ).wait()

                    pltpu.async_copy(table_hbm.at[idx_n], val_n, g_n)

                pltpu.make_async_copy(table_hbm.at[idx_c], val_c, g_c).wait()
                pltpu.async_copy(
                    val_c, out_hbm.at[it, pl.ds(base, per_w)], o_c
                )

        pltpu.make_async_copy(
            val0, out_hbm.at[jnp.int32(n_it - 2), pl.ds(base, per_w)], o0
        ).wait()
        pltpu.make_async_copy(
            val1, out_hbm.at[jnp.int32(n_it - 1), pl.ds(base, per_w)], o1
        ).wait()

    return k(table_1d, idx)


def _mlp_body(x_ref, w1_ref, b1_ref, w2_ref, b2_ref, w3_ref, b3_ref, o_ref):
    x = x_ref[...]
    h = jnp.dot(w1_ref[...], x, preferred_element_type=jnp.float32) + b1_ref[...]
    h = jnp.maximum(h, 0.0)
    h = jnp.dot(w2_ref[...], h, preferred_element_type=jnp.float32) + b2_ref[...]
    h = jnp.maximum(h, 0.0)
    o = jnp.dot(w3_ref[...], h, preferred_element_type=jnp.float32) + b3_ref[...]
    sig = 1.0 / (1.0 + jnp.exp(-o))
    sp = jnp.maximum(o, 0.0) + jnp.log(1.0 + jnp.exp(-jnp.abs(o)))
    row = lax.broadcasted_iota(jnp.int32, o.shape, 0)
    o_ref[...] = jnp.where(row < 3, sig, sp)


def _mlp(feats, W1t, b1, W2t, b2, W3t, b3, n):
    blk = 8192
    grid = n // blk
    full = lambda a: pl.BlockSpec(a.shape, lambda i: tuple(i * 0 for _ in a.shape))
    return pl.pallas_call(
        _mlp_body,
        grid=(grid,),
        in_specs=[
            pl.BlockSpec((MLP_IN, blk), lambda i: (i * 0, i)),
            full(W1t), full(b1), full(W2t), full(b2), full(W3t), full(b3),
        ],
        out_specs=pl.BlockSpec((4, blk), lambda i: (i * 0, i)),
        out_shape=jax.ShapeDtypeStruct((4, n), jnp.float32),
    )(feats, W1t, b1, W2t, b2, W3t, b3)


def kernel(positions, tables, W1, b1, W2, b2, W3, b3):
    f32 = jnp.float32
    positions = positions.astype(f32)
    tables = tables.astype(f32)
    W1, b1, W2, b2, W3, b3 = (a.astype(f32) for a in (W1, b1, W2, b2, W3, b3))

    table_1d = (
        tables.reshape(NUM_LEVELS, TABLE_SIZE // 128, 128, F_PER_LEVEL)
        .transpose(0, 1, 3, 2)
        .reshape(-1)
    )
    pos_t = positions.T
    w_args = (W1.T, b1.reshape(HIDDEN, 1), W2.T, b2.reshape(HIDDEN, 1),
              W3.T, b3.reshape(4, 1))
    ns = _NSLICE
    bs = BATCH // ns
    ys = []
    for s in range(ns):
        sl = slice(s * bs, (s + 1) * bs)
        idx_s = _compute_indices(pos_t[:, sl], bs)
        feats_s = _gather_sc(table_1d, idx_s, bs)
        ys.append(_mlp(feats_s, *w_args, bs))
    y = jnp.concatenate(ys, axis=1) if ns > 1 else ys[0]
    yt = y.T.astype(jnp.float64)
    return yt[:, :3], yt[:, 3:4]


# speedup vs baseline: 82.3291x; 1.0455x over previous
import functools

import jax
import jax.numpy as jnp
from jax import lax
from jax.experimental import pallas as pl
from jax.experimental.pallas import tpu as pltpu
from jax.experimental.pallas import tpu_sc as plsc

NUM_LEVELS = 16
F_PER_LEVEL = 2
LOG2_HASHMAP = 19
TABLE_SIZE = 2 ** LOG2_HASHMAP
BATCH = 262144
HIDDEN = 64
MLP_IN = NUM_LEVELS * F_PER_LEVEL

_RES = [int(16 * (2048 / 16) ** (l / (NUM_LEVELS - 1))) for l in range(NUM_LEVELS)]
_M1 = 2654435761 - (1 << 32)
_M2 = 805459861

_NW = 32
_NSLICE = 4


def _hash_body(pos_ref, res_ref, idx_ref):
    p = pos_ref[...]
    res = res_ref[...]
    r = lax.broadcasted_iota(jnp.int32, (32, 1), 0)
    lvl = r >> 1
    feat = r & 1
    c0 = (p[0:1, :] * res).astype(jnp.int32)
    c1 = (p[1:2, :] * res).astype(jnp.int32)
    c2 = (p[2:3, :] * res).astype(jnp.int32)
    h = c0 ^ (c1 * jnp.int32(_M1)) ^ (c2 * jnp.int32(_M2))
    e = h & jnp.int32(TABLE_SIZE - 1)
    idx_ref[...] = (
        (lvl << 20)
        + ((e >> 7) << 8)
        + (feat << 7)
        + (e & jnp.int32(127))
    )


def _compute_indices(pos_t, n):
    blk = 8192
    grid = n // blk
    res = jnp.repeat(jnp.array(_RES, dtype=jnp.float32), 2).reshape(32, 1)
    return pl.pallas_call(
        _hash_body,
        grid=(grid,),
        in_specs=[
            pl.BlockSpec((3, blk), lambda i: (i * 0, i)),
            pl.BlockSpec((32, 1), lambda i: (i * 0, i * 0)),
        ],
        out_specs=pl.BlockSpec((32, blk), lambda i: (i * 0, i)),
        out_shape=jax.ShapeDtypeStruct((MLP_IN, n), jnp.int32),
    )(pos_t, res)


def _gather_sc(table_1d, idx, n):
    per_w = n // _NW
    n_it = MLP_IN
    mesh = plsc.VectorSubcoreMesh(core_axis_name="c", subcore_axis_name="s")

    nbuf = 4

    @functools.partial(
        pl.kernel,
        out_type=jax.ShapeDtypeStruct((MLP_IN, n), jnp.float32),
        mesh=mesh,
        compiler_params=pltpu.CompilerParams(use_tc_tiling_on_sc=False),
        scratch_types=[pltpu.VMEM((per_w,), jnp.int32) for _ in range(nbuf)]
        + [pltpu.VMEM((per_w,), jnp.float32) for _ in range(nbuf)]
        + [pltpu.SemaphoreType.DMA for _ in range(2 * nbuf)],
    )
    def k(table_hbm, idx_hbm, out_hbm, *bufs):
        idxb = bufs[:nbuf]
        valb = bufs[nbuf:2 * nbuf]
        gsem = bufs[2 * nbuf:3 * nbuf]
        osem = bufs[3 * nbuf:]
        wid = (lax.axis_index("s") * 2 + lax.axis_index("c")).astype(jnp.int32)
        base = wid * jnp.int32(per_w)

        for j in range(nbuf - 1):
            pltpu.sync_copy(idx_hbm.at[jnp.int32(j), pl.ds(base, per_w)], idxb[j])
            pltpu.async_copy(table_hbm.at[idxb[j]], valb[j], gsem[j])

        @pl.loop(jnp.int32(0), jnp.int32(n_it), step=nbuf)
        def body(it0):
            for b in range(nbuf):
                it = it0 + jnp.int32(b)
                sn = (b + nbuf - 1) % nbuf
                j = it + jnp.int32(nbuf - 1)

                @pl.when(j < n_it)
                def _():
                    pltpu.sync_copy(idx_hbm.at[j, pl.ds(base, per_w)], idxb[sn])
                    @pl.when(j >= nbuf)
                    def _():
                        pltpu.make_async_copy(
                            valb[sn],
                            out_hbm.at[j - nbuf, pl.ds(base, per_w)],
                            osem[sn],
                        ).wait()

                    pltpu.async_copy(table_hbm.at[idxb[sn]], valb[sn], gsem[sn])

                pltpu.make_async_copy(
                    table_hbm.at[idxb[b]], valb[b], gsem[b]
                ).wait()
                pltpu.async_copy(
                    valb[b], out_hbm.at[it, pl.ds(base, per_w)], osem[b]
                )

        for j in range(nbuf):
            r = n_it - nbuf + j
            pltpu.make_async_copy(
                valb[r % nbuf], out_hbm.at[jnp.int32(r), pl.ds(base, per_w)],
                osem[r % nbuf],
            ).wait()

    return k(table_1d, idx)


def _mlp_body(x_ref, w1_ref, b1_ref, w2_ref, b2_ref, w3_ref, b3_ref, o_ref):
    x = x_ref[...]
    h = jnp.dot(w1_ref[...], x, preferred_element_type=jnp.float32) + b1_ref[...]
    h = jnp.maximum(h, 0.0)
    h = jnp.dot(w2_ref[...], h, preferred_element_type=jnp.float32) + b2_ref[...]
    h = jnp.maximum(h, 0.0)
    o = jnp.dot(w3_ref[...], h, preferred_element_type=jnp.float32) + b3_ref[...]
    sig = 1.0 / (1.0 + jnp.exp(-o))
    sp = jnp.maximum(o, 0.0) + jnp.log(1.0 + jnp.exp(-jnp.abs(o)))
    row = lax.broadcasted_iota(jnp.int32, o.shape, 0)
    o_ref[...] = jnp.where(row < 3, sig, sp)


def _mlp(feats, W1t, b1, W2t, b2, W3t, b3, n):
    blk = 8192
    grid = n // blk
    full = lambda a: pl.BlockSpec(a.shape, lambda i: tuple(i * 0 for _ in a.shape))
    return pl.pallas_call(
        _mlp_body,
        grid=(grid,),
        in_specs=[
            pl.BlockSpec((MLP_IN, blk), lambda i: (i * 0, i)),
            full(W1t), full(b1), full(W2t), full(b2), full(W3t), full(b3),
        ],
        out_specs=pl.BlockSpec((4, blk), lambda i: (i * 0, i)),
        out_shape=jax.ShapeDtypeStruct((4, n), jnp.float32),
    )(feats, W1t, b1, W2t, b2, W3t, b3)


def kernel(positions, tables, W1, b1, W2, b2, W3, b3):
    f32 = jnp.float32
    positions = positions.astype(f32)
    tables = tables.astype(f32)
    W1, b1, W2, b2, W3, b3 = (a.astype(f32) for a in (W1, b1, W2, b2, W3, b3))

    table_1d = (
        tables.reshape(NUM_LEVELS, TABLE_SIZE // 128, 128, F_PER_LEVEL)
        .transpose(0, 1, 3, 2)
        .reshape(-1)
    )
    pos_t = positions.T
    w_args = (W1.T, b1.reshape(HIDDEN, 1), W2.T, b2.reshape(HIDDEN, 1),
              W3.T, b3.reshape(4, 1))
    ns = _NSLICE
    bs = BATCH // ns
    ys = []
    for s in range(ns):
        sl = slice(s * bs, (s + 1) * bs)
        idx_s = _compute_indices(pos_t[:, sl], bs)
        feats_s = _gather_sc(table_1d, idx_s, bs)
        ys.append(_mlp(feats_s, *w_args, bs))
    y = jnp.concatenate(ys, axis=1) if ns > 1 else ys[0]
    yt = y.T.astype(jnp.float64)
    return yt[:, :3], yt[:, 3:4]


# speedup vs baseline: 82.4123x; 1.0010x over previous
import functools

import jax
import jax.numpy as jnp
from jax import lax
from jax.experimental import pallas as pl
from jax.experimental.pallas import tpu as pltpu
from jax.experimental.pallas import tpu_sc as plsc

NUM_LEVELS = 16
F_PER_LEVEL = 2
LOG2_HASHMAP = 19
TABLE_SIZE = 2 ** LOG2_HASHMAP
BATCH = 262144
HIDDEN = 64
MLP_IN = NUM_LEVELS * F_PER_LEVEL

_RES = [int(16 * (2048 / 16) ** (l / (NUM_LEVELS - 1))) for l in range(NUM_LEVELS)]
_M1 = 2654435761 - (1 << 32)
_M2 = 805459861

_NW = 32
_NSLICE = 4


def _hash_body(pos_ref, res_ref, idx_ref):
    p = pos_ref[...]
    res = res_ref[...]
    r = lax.broadcasted_iota(jnp.int32, (32, 1), 0)
    lvl = r >> 1
    feat = r & 1
    c0 = (p[0:1, :] * res).astype(jnp.int32)
    c1 = (p[1:2, :] * res).astype(jnp.int32)
    c2 = (p[2:3, :] * res).astype(jnp.int32)
    h = c0 ^ (c1 * jnp.int32(_M1)) ^ (c2 * jnp.int32(_M2))
    e = h & jnp.int32(TABLE_SIZE - 1)
    idx_ref[...] = (
        (lvl << 20)
        + ((e >> 7) << 8)
        + (feat << 7)
        + (e & jnp.int32(127))
    )


def _compute_indices(pos_t, n):
    blk = 8192
    grid = n // blk
    res = jnp.repeat(jnp.array(_RES, dtype=jnp.float32), 2).reshape(32, 1)
    return pl.pallas_call(
        _hash_body,
        grid=(grid,),
        in_specs=[
            pl.BlockSpec((3, blk), lambda i: (i * 0, i)),
            pl.BlockSpec((32, 1), lambda i: (i * 0, i * 0)),
        ],
        out_specs=pl.BlockSpec((32, blk), lambda i: (i * 0, i)),
        out_shape=jax.ShapeDtypeStruct((MLP_IN, n), jnp.int32),
    )(pos_t, res)


def _gather_sc(table_1d, idx, n):
    per_w = n // _NW
    n_it = MLP_IN
    mesh = plsc.VectorSubcoreMesh(core_axis_name="c", subcore_axis_name="s")

    nbuf = 8

    @functools.partial(
        pl.kernel,
        out_type=jax.ShapeDtypeStruct((MLP_IN, n), jnp.float32),
        mesh=mesh,
        compiler_params=pltpu.CompilerParams(use_tc_tiling_on_sc=False),
        scratch_types=[pltpu.VMEM((per_w,), jnp.int32) for _ in range(nbuf)]
        + [pltpu.VMEM((per_w,), jnp.float32) for _ in range(nbuf)]
        + [pltpu.SemaphoreType.DMA for _ in range(2 * nbuf)],
    )
    def k(table_hbm, idx_hbm, out_hbm, *bufs):
        idxb = bufs[:nbuf]
        valb = bufs[nbuf:2 * nbuf]
        gsem = bufs[2 * nbuf:3 * nbuf]
        osem = bufs[3 * nbuf:]
        wid = (lax.axis_index("s") * 2 + lax.axis_index("c")).astype(jnp.int32)
        base = wid * jnp.int32(per_w)

        for j in range(nbuf - 1):
            pltpu.sync_copy(idx_hbm.at[jnp.int32(j), pl.ds(base, per_w)], idxb[j])
            pltpu.async_copy(table_hbm.at[idxb[j]], valb[j], gsem[j])

        @pl.loop(jnp.int32(0), jnp.int32(n_it), step=nbuf)
        def body(it0):
            for b in range(nbuf):
                it = it0 + jnp.int32(b)
                sn = (b + nbuf - 1) % nbuf
                j = it + jnp.int32(nbuf - 1)

                @pl.when(j < n_it)
                def _():
                    pltpu.sync_copy(idx_hbm.at[j, pl.ds(base, per_w)], idxb[sn])
                    @pl.when(j >= nbuf)
                    def _():
                        pltpu.make_async_copy(
                            valb[sn],
                            out_hbm.at[j - nbuf, pl.ds(base, per_w)],
                            osem[sn],
                        ).wait()

                    pltpu.async_copy(table_hbm.at[idxb[sn]], valb[sn], gsem[sn])

                pltpu.make_async_copy(
                    table_hbm.at[idxb[b]], valb[b], gsem[b]
                ).wait()
                pltpu.async_copy(
                    valb[b], out_hbm.at[it, pl.ds(base, per_w)], osem[b]
                )

        for j in range(nbuf):
            r = n_it - nbuf + j
            pltpu.make_async_copy(
                valb[r % nbuf], out_hbm.at[jnp.int32(r), pl.ds(base, per_w)],
                osem[r % nbuf],
            ).wait()

    return k(table_1d, idx)


def _mlp_body(x_ref, w1_ref, b1_ref, w2_ref, b2_ref, w3_ref, b3_ref, o_ref):
    x = x_ref[...]
    h = jnp.dot(w1_ref[...], x, preferred_element_type=jnp.float32) + b1_ref[...]
    h = jnp.maximum(h, 0.0)
    h = jnp.dot(w2_ref[...], h, preferred_element_type=jnp.float32) + b2_ref[...]
    h = jnp.maximum(h, 0.0)
    o = jnp.dot(w3_ref[...], h, preferred_element_type=jnp.float32) + b3_ref[...]
    sig = 1.0 / (1.0 + jnp.exp(-o))
    sp = jnp.maximum(o, 0.0) + jnp.log(1.0 + jnp.exp(-jnp.abs(o)))
    row = lax.broadcasted_iota(jnp.int32, o.shape, 0)
    o_ref[...] = jnp.where(row < 3, sig, sp)


def _mlp(feats, W1t, b1, W2t, b2, W3t, b3, n):
    blk = 8192
    grid = n // blk
    full = lambda a: pl.BlockSpec(a.shape, lambda i: tuple(i * 0 for _ in a.shape))
    return pl.pallas_call(
        _mlp_body,
        grid=(grid,),
        in_specs=[
            pl.BlockSpec((MLP_IN, blk), lambda i: (i * 0, i)),
            full(W1t), full(b1), full(W2t), full(b2), full(W3t), full(b3),
        ],
        out_specs=pl.BlockSpec((4, blk), lambda i: (i * 0, i)),
        out_shape=jax.ShapeDtypeStruct((4, n), jnp.float32),
    )(feats, W1t, b1, W2t, b2, W3t, b3)


def kernel(positions, tables, W1, b1, W2, b2, W3, b3):
    f32 = jnp.float32
    positions = positions.astype(f32)
    tables = tables.astype(f32)
    W1, b1, W2, b2, W3, b3 = (a.astype(f32) for a in (W1, b1, W2, b2, W3, b3))

    table_1d = (
        tables.reshape(NUM_LEVELS, TABLE_SIZE // 128, 128, F_PER_LEVEL)
        .transpose(0, 1, 3, 2)
        .reshape(-1)
    )
    pos_t = positions.T
    w_args = (W1.T, b1.reshape(HIDDEN, 1), W2.T, b2.reshape(HIDDEN, 1),
              W3.T, b3.reshape(4, 1))
    ns = _NSLICE
    bs = BATCH // ns
    ys = []
    for s in range(ns):
        sl = slice(s * bs, (s + 1) * bs)
        idx_s = _compute_indices(pos_t[:, sl], bs)
        feats_s = _gather_sc(table_1d, idx_s, bs)
        ys.append(_mlp(feats_s, *w_args, bs))
    y = jnp.concatenate(ys, axis=1) if ns > 1 else ys[0]
    yt = y.T.astype(jnp.float64)
    return yt[:, :3], yt[:, 3:4]


# speedup vs baseline: 90.3190x; 1.0959x over previous
import functools

import jax
import jax.numpy as jnp
from jax import lax
from jax.experimental import pallas as pl
from jax.experimental.pallas import tpu as pltpu
from jax.experimental.pallas import tpu_sc as plsc

NUM_LEVELS = 16
F_PER_LEVEL = 2
LOG2_HASHMAP = 19
TABLE_SIZE = 2 ** LOG2_HASHMAP
BATCH = 262144
HIDDEN = 64
MLP_IN = NUM_LEVELS * F_PER_LEVEL

_RES = [int(16 * (2048 / 16) ** (l / (NUM_LEVELS - 1))) for l in range(NUM_LEVELS)]
_M1 = 2654435761 - (1 << 32)
_M2 = 805459861

_NW = 32
_NSLICE = 4


def _hash_body(pos_ref, res_ref, idx_ref):
    p = pos_ref[...]
    res = res_ref[...]
    r = lax.broadcasted_iota(jnp.int32, (32, 1), 0)
    lvl = r >> 1
    feat = r & 1
    c0 = (p[0:1, :] * res).astype(jnp.int32)
    c1 = (p[1:2, :] * res).astype(jnp.int32)
    c2 = (p[2:3, :] * res).astype(jnp.int32)
    h = c0 ^ (c1 * jnp.int32(_M1)) ^ (c2 * jnp.int32(_M2))
    e = h & jnp.int32(TABLE_SIZE - 1)
    idx_ref[...] = (
        (lvl << 20)
        + ((e >> 7) << 8)
        + (feat << 7)
        + (e & jnp.int32(127))
    )


def _compute_indices(pos_t, n):
    blk = 8192
    grid = n // blk
    res = jnp.repeat(jnp.array(_RES, dtype=jnp.float32), 2).reshape(32, 1)
    return pl.pallas_call(
        _hash_body,
        grid=(grid,),
        in_specs=[
            pl.BlockSpec((3, blk), lambda i: (i * 0, i)),
            pl.BlockSpec((32, 1), lambda i: (i * 0, i * 0)),
        ],
        out_specs=pl.BlockSpec((32, blk), lambda i: (i * 0, i)),
        out_shape=jax.ShapeDtypeStruct((MLP_IN, n), jnp.int32),
    )(pos_t, res)


def _gather_sc(table_1d, idx_flat, total):
    per_w = total // _NW
    ch = 8192
    n_it = per_w // ch
    mesh = plsc.VectorSubcoreMesh(core_axis_name="c", subcore_axis_name="s")

    nbuf = min(4, n_it)

    @functools.partial(
        pl.kernel,
        out_type=jax.ShapeDtypeStruct((total,), jnp.float32),
        mesh=mesh,
        compiler_params=pltpu.CompilerParams(use_tc_tiling_on_sc=False),
        scratch_types=[pltpu.VMEM((ch,), jnp.int32) for _ in range(nbuf)]
        + [pltpu.VMEM((ch,), jnp.float32) for _ in range(nbuf)]
        + [pltpu.SemaphoreType.DMA for _ in range(2 * nbuf)],
    )
    def k(table_hbm, idx_hbm, out_hbm, *bufs):
        idxb = bufs[:nbuf]
        valb = bufs[nbuf:2 * nbuf]
        gsem = bufs[2 * nbuf:3 * nbuf]
        osem = bufs[3 * nbuf:]
        wid = (lax.axis_index("s") * 2 + lax.axis_index("c")).astype(jnp.int32)
        base = wid * jnp.int32(per_w)

        for j in range(nbuf - 1):
            pltpu.sync_copy(
                idx_hbm.at[pl.ds(base + jnp.int32(j * ch), ch)], idxb[j]
            )
            pltpu.async_copy(table_hbm.at[idxb[j]], valb[j], gsem[j])

        @pl.loop(jnp.int32(0), jnp.int32(n_it), step=nbuf)
        def body(it0):
            for b in range(nbuf):
                it = it0 + jnp.int32(b)
                sn = (b + nbuf - 1) % nbuf
                j = it + jnp.int32(nbuf - 1)

                @pl.when(j < n_it)
                def _():
                    pltpu.sync_copy(
                        idx_hbm.at[pl.ds(base + j * ch, ch)], idxb[sn]
                    )
                    @pl.when(j >= nbuf)
                    def _():
                        pltpu.make_async_copy(
                            valb[sn],
                            out_hbm.at[pl.ds(base + (j - nbuf) * ch, ch)],
                            osem[sn],
                        ).wait()

                    pltpu.async_copy(table_hbm.at[idxb[sn]], valb[sn], gsem[sn])

                pltpu.make_async_copy(
                    table_hbm.at[idxb[b]], valb[b], gsem[b]
                ).wait()
                pltpu.async_copy(
                    valb[b], out_hbm.at[pl.ds(base + it * ch, ch)], osem[b]
                )

        for j in range(nbuf):
            r = n_it - nbuf + j
            pltpu.make_async_copy(
                valb[r % nbuf],
                out_hbm.at[pl.ds(base + jnp.int32(r * ch), ch)],
                osem[r % nbuf],
            ).wait()

    return k(table_1d, idx_flat)


def _mlp_body(x_ref, w1_ref, b1_ref, w2_ref, b2_ref, w3_ref, b3_ref, o_ref):
    x = x_ref[...]
    h = jnp.dot(w1_ref[...], x, preferred_element_type=jnp.float32) + b1_ref[...]
    h = jnp.maximum(h, 0.0)
    h = jnp.dot(w2_ref[...], h, preferred_element_type=jnp.float32) + b2_ref[...]
    h = jnp.maximum(h, 0.0)
    o = jnp.dot(w3_ref[...], h, preferred_element_type=jnp.float32) + b3_ref[...]
    sig = 1.0 / (1.0 + jnp.exp(-o))
    sp = jnp.maximum(o, 0.0) + jnp.log(1.0 + jnp.exp(-jnp.abs(o)))
    row = lax.broadcasted_iota(jnp.int32, o.shape, 0)
    o_ref[...] = jnp.where(row < 3, sig, sp)


def _mlp(feats, W1t, b1, W2t, b2, W3t, b3, n):
    blk = 8192
    grid = n // blk
    full = lambda a: pl.BlockSpec(a.shape, lambda i: tuple(i * 0 for _ in a.shape))
    return pl.pallas_call(
        _mlp_body,
        grid=(grid,),
        in_specs=[
            pl.BlockSpec((MLP_IN, blk), lambda i: (i * 0, i)),
            full(W1t), full(b1), full(W2t), full(b2), full(W3t), full(b3),
        ],
        out_specs=pl.BlockSpec((4, blk), lambda i: (i * 0, i)),
        out_shape=jax.ShapeDtypeStruct((4, n), jnp.float32),
    )(feats, W1t, b1, W2t, b2, W3t, b3)


def kernel(positions, tables, W1, b1, W2, b2, W3, b3):
    f32 = jnp.float32
    positions = positions.astype(f32)
    tables = tables.astype(f32)
    W1, b1, W2, b2, W3, b3 = (a.astype(f32) for a in (W1, b1, W2, b2, W3, b3))

    table_1d = (
        tables.reshape(NUM_LEVELS, TABLE_SIZE // 128, 128, F_PER_LEVEL)
        .transpose(0, 1, 3, 2)
        .reshape(-1)
    )
    pos_t = positions.T
    w_args = (W1.T, b1.reshape(HIDDEN, 1), W2.T, b2.reshape(HIDDEN, 1),
              W3.T, b3.reshape(4, 1))
    ns = _NSLICE
    bs = BATCH // ns
    ys = []
    for s in range(ns):
        sl = slice(s * bs, (s + 1) * bs)
        idx_s = _compute_indices(pos_t[:, sl], bs)
        idx_flat = (
            idx_s.reshape(4, 8, bs // 128, 128).transpose(0, 2, 1, 3).reshape(-1)
        )
        out_flat = _gather_sc(table_1d, idx_flat, MLP_IN * bs)
        feats_s = (
            out_flat.reshape(4, bs // 128, 8, 128)
            .transpose(0, 2, 1, 3)
            .reshape(MLP_IN, bs)
        )
        ys.append(_mlp(feats_s, *w_args, bs))
    y = jnp.concatenate(ys, axis=1) if ns > 1 else ys[0]
    yt = y.T.astype(jnp.float64)
    return yt[:, :3], yt[:, 3:4]


# speedup vs baseline: 90.4967x; 1.0020x over previous
import functools

import jax
import jax.numpy as jnp
from jax import lax
from jax.experimental import pallas as pl
from jax.experimental.pallas import tpu as pltpu
from jax.experimental.pallas import tpu_sc as plsc

NUM_LEVELS = 16
F_PER_LEVEL = 2
LOG2_HASHMAP = 19
TABLE_SIZE = 2 ** LOG2_HASHMAP
BATCH = 262144
HIDDEN = 64
MLP_IN = NUM_LEVELS * F_PER_LEVEL

_RES = [int(16 * (2048 / 16) ** (l / (NUM_LEVELS - 1))) for l in range(NUM_LEVELS)]
_M1 = 2654435761 - (1 << 32)
_M2 = 805459861

_NW = 32
_NSLICE = 8


def _hash_body(pos_ref, res_ref, idx_ref):
    p = pos_ref[...]
    res = res_ref[...]
    r = lax.broadcasted_iota(jnp.int32, (32, 1), 0)
    lvl = r >> 1
    feat = r & 1
    c0 = (p[0:1, :] * res).astype(jnp.int32)
    c1 = (p[1:2, :] * res).astype(jnp.int32)
    c2 = (p[2:3, :] * res).astype(jnp.int32)
    h = c0 ^ (c1 * jnp.int32(_M1)) ^ (c2 * jnp.int32(_M2))
    e = h & jnp.int32(TABLE_SIZE - 1)
    idx_ref[...] = (
        (lvl << 20)
        + ((e >> 7) << 8)
        + (feat << 7)
        + (e & jnp.int32(127))
    )


def _compute_indices(pos_t, n):
    blk = 8192
    grid = n // blk
    res = jnp.repeat(jnp.array(_RES, dtype=jnp.float32), 2).reshape(32, 1)
    return pl.pallas_call(
        _hash_body,
        grid=(grid,),
        in_specs=[
            pl.BlockSpec((3, blk), lambda i: (i * 0, i)),
            pl.BlockSpec((32, 1), lambda i: (i * 0, i * 0)),
        ],
        out_specs=pl.BlockSpec((32, blk), lambda i: (i * 0, i)),
        out_shape=jax.ShapeDtypeStruct((MLP_IN, n), jnp.int32),
    )(pos_t, res)


def _gather_sc(table_1d, idx_flat, total):
    per_w = total // _NW
    ch = 8192
    n_it = per_w // ch
    mesh = plsc.VectorSubcoreMesh(core_axis_name="c", subcore_axis_name="s")

    nbuf = min(4, n_it)

    @functools.partial(
        pl.kernel,
        out_type=jax.ShapeDtypeStruct((total,), jnp.float32),
        mesh=mesh,
        compiler_params=pltpu.CompilerParams(use_tc_tiling_on_sc=False),
        scratch_types=[pltpu.VMEM((ch,), jnp.int32) for _ in range(nbuf)]
        + [pltpu.VMEM((ch,), jnp.float32) for _ in range(nbuf)]
        + [pltpu.SemaphoreType.DMA for _ in range(2 * nbuf)],
    )
    def k(table_hbm, idx_hbm, out_hbm, *bufs):
        idxb = bufs[:nbuf]
        valb = bufs[nbuf:2 * nbuf]
        gsem = bufs[2 * nbuf:3 * nbuf]
        osem = bufs[3 * nbuf:]
        wid = (lax.axis_index("s") * 2 + lax.axis_index("c")).astype(jnp.int32)
        base = wid * jnp.int32(per_w)

        for j in range(nbuf - 1):
            pltpu.sync_copy(
                idx_hbm.at[pl.ds(base + jnp.int32(j * ch), ch)], idxb[j]
            )
            pltpu.async_copy(table_hbm.at[idxb[j]], valb[j], gsem[j])

        @pl.loop(jnp.int32(0), jnp.int32(n_it), step=nbuf)
        def body(it0):
            for b in range(nbuf):
                it = it0 + jnp.int32(b)
                sn = (b + nbuf - 1) % nbuf
                j = it + jnp.int32(nbuf - 1)

                @pl.when(j < n_it)
                def _():
                    pltpu.sync_copy(
                        idx_hbm.at[pl.ds(base + j * ch, ch)], idxb[sn]
                    )
                    @pl.when(j >= nbuf)
                    def _():
                        pltpu.make_async_copy(
                            valb[sn],
                            out_hbm.at[pl.ds(base + (j - nbuf) * ch, ch)],
                            osem[sn],
                        ).wait()

                    pltpu.async_copy(table_hbm.at[idxb[sn]], valb[sn], gsem[sn])

                pltpu.make_async_copy(
                    table_hbm.at[idxb[b]], valb[b], gsem[b]
                ).wait()
                pltpu.async_copy(
                    valb[b], out_hbm.at[pl.ds(base + it * ch, ch)], osem[b]
                )

        for j in range(nbuf):
            r = n_it - nbuf + j
            pltpu.make_async_copy(
                valb[r % nbuf],
                out_hbm.at[pl.ds(base + jnp.int32(r * ch), ch)],
                osem[r % nbuf],
            ).wait()

    return k(table_1d, idx_flat)


def _mlp_body(x_ref, w1_ref, b1_ref, w2_ref, b2_ref, w3_ref, b3_ref, o_ref):
    x = x_ref[...]
    h = jnp.dot(w1_ref[...], x, preferred_element_type=jnp.float32) + b1_ref[...]
    h = jnp.maximum(h, 0.0)
    h = jnp.dot(w2_ref[...], h, preferred_element_type=jnp.float32) + b2_ref[...]
    h = jnp.maximum(h, 0.0)
    o = jnp.dot(w3_ref[...], h, preferred_element_type=jnp.float32) + b3_ref[...]
    sig = 1.0 / (1.0 + jnp.exp(-o))
    sp = jnp.maximum(o, 0.0) + jnp.log(1.0 + jnp.exp(-jnp.abs(o)))
    row = lax.broadcasted_iota(jnp.int32, o.shape, 0)
    o_ref[...] = jnp.where(row < 3, sig, sp)


def _mlp(feats, W1t, b1, W2t, b2, W3t, b3, n):
    blk = 8192
    grid = n // blk
    full = lambda a: pl.BlockSpec(a.shape, lambda i: tuple(i * 0 for _ in a.shape))
    return pl.pallas_call(
        _mlp_body,
        grid=(grid,),
        in_specs=[
            pl.BlockSpec((MLP_IN, blk), lambda i: (i * 0, i)),
            full(W1t), full(b1), full(W2t), full(b2), full(W3t), full(b3),
        ],
        out_specs=pl.BlockSpec((4, blk), lambda i: (i * 0, i)),
        out_shape=jax.ShapeDtypeStruct((4, n), jnp.float32),
    )(feats, W1t, b1, W2t, b2, W3t, b3)


def kernel(positions, tables, W1, b1, W2, b2, W3, b3):
    f32 = jnp.float32
    positions = positions.astype(f32)
    tables = tables.astype(f32)
    W1, b1, W2, b2, W3, b3 = (a.astype(f32) for a in (W1, b1, W2, b2, W3, b3))

    table_1d = (
        tables.reshape(NUM_LEVELS, TABLE_SIZE // 128, 128, F_PER_LEVEL)
        .transpose(0, 1, 3, 2)
        .reshape(-1)
    )
    pos_t = positions.T
    w_args = (W1.T, b1.reshape(HIDDEN, 1), W2.T, b2.reshape(HIDDEN, 1),
              W3.T, b3.reshape(4, 1))
    ns = _NSLICE
    bs = BATCH // ns
    ys = []
    for s in range(ns):
        sl = slice(s * bs, (s + 1) * bs)
        idx_s = _compute_indices(pos_t[:, sl], bs)
        idx_flat = (
            idx_s.reshape(4, 8, bs // 128, 128).transpose(0, 2, 1, 3).reshape(-1)
        )
        out_flat = _gather_sc(table_1d, idx_flat, MLP_IN * bs)
        feats_s = (
            out_flat.reshape(4, bs // 128, 8, 128)
            .transpose(0, 2, 1, 3)
            .reshape(MLP_IN, bs)
        )
        ys.append(_mlp(feats_s, *w_args, bs))
    y = jnp.concatenate(ys, axis=1) if ns > 1 else ys[0]
    yt = y.T.astype(jnp.float64)
    return yt[:, :3], yt[:, 3:4]


# speedup vs baseline: 91.3969x; 1.0099x over previous
import functools

import jax
import jax.numpy as jnp
from jax import lax
from jax.experimental import pallas as pl
from jax.experimental.pallas import tpu as pltpu
from jax.experimental.pallas import tpu_sc as plsc

NUM_LEVELS = 16
F_PER_LEVEL = 2
LOG2_HASHMAP = 19
TABLE_SIZE = 2 ** LOG2_HASHMAP
BATCH = 262144
HIDDEN = 64
MLP_IN = NUM_LEVELS * F_PER_LEVEL

_RES = [int(16 * (2048 / 16) ** (l / (NUM_LEVELS - 1))) for l in range(NUM_LEVELS)]
_M1 = 2654435761 - (1 << 32)
_M2 = 805459861

_NW = 32
_NSLICE = 2


def _hash_body(pos_ref, res_ref, idx_ref):
    p = pos_ref[...]
    res = res_ref[...]
    r = lax.broadcasted_iota(jnp.int32, (32, 1), 0)
    lvl = r >> 1
    feat = r & 1
    c0 = (p[0:1, :] * res).astype(jnp.int32)
    c1 = (p[1:2, :] * res).astype(jnp.int32)
    c2 = (p[2:3, :] * res).astype(jnp.int32)
    h = c0 ^ (c1 * jnp.int32(_M1)) ^ (c2 * jnp.int32(_M2))
    e = h & jnp.int32(TABLE_SIZE - 1)
    idx_ref[...] = (
        (lvl << 20)
        + ((e >> 7) << 8)
        + (feat << 7)
        + (e & jnp.int32(127))
    )


def _compute_indices(pos_t, n):
    blk = 8192
    grid = n // blk
    res = jnp.repeat(jnp.array(_RES, dtype=jnp.float32), 2).reshape(32, 1)
    return pl.pallas_call(
        _hash_body,
        grid=(grid,),
        in_specs=[
            pl.BlockSpec((3, blk), lambda i: (i * 0, i)),
            pl.BlockSpec((32, 1), lambda i: (i * 0, i * 0)),
        ],
        out_specs=pl.BlockSpec((32, blk), lambda i: (i * 0, i)),
        out_shape=jax.ShapeDtypeStruct((MLP_IN, n), jnp.int32),
    )(pos_t, res)


def _gather_sc(table_1d, idx_flat, total):
    per_w = total // _NW
    ch = 8192
    n_it = per_w // ch
    mesh = plsc.VectorSubcoreMesh(core_axis_name="c", subcore_axis_name="s")

    nbuf = min(4, n_it)

    @functools.partial(
        pl.kernel,
        out_type=jax.ShapeDtypeStruct((total,), jnp.float32),
        mesh=mesh,
        compiler_params=pltpu.CompilerParams(use_tc_tiling_on_sc=False),
        scratch_types=[pltpu.VMEM((ch,), jnp.int32) for _ in range(nbuf)]
        + [pltpu.VMEM((ch,), jnp.float32) for _ in range(nbuf)]
        + [pltpu.SemaphoreType.DMA for _ in range(2 * nbuf)],
    )
    def k(table_hbm, idx_hbm, out_hbm, *bufs):
        idxb = bufs[:nbuf]
        valb = bufs[nbuf:2 * nbuf]
        gsem = bufs[2 * nbuf:3 * nbuf]
        osem = bufs[3 * nbuf:]
        wid = (lax.axis_index("s") * 2 + lax.axis_index("c")).astype(jnp.int32)
        base = wid * jnp.int32(per_w)

        for j in range(nbuf - 1):
            pltpu.sync_copy(
                idx_hbm.at[pl.ds(base + jnp.int32(j * ch), ch)], idxb[j]
            )
            pltpu.async_copy(table_hbm.at[idxb[j]], valb[j], gsem[j])

        @pl.loop(jnp.int32(0), jnp.int32(n_it), step=nbuf)
        def body(it0):
            for b in range(nbuf):
                it = it0 + jnp.int32(b)
                sn = (b + nbuf - 1) % nbuf
                j = it + jnp.int32(nbuf - 1)

                @pl.when(j < n_it)
                def _():
                    pltpu.sync_copy(
                        idx_hbm.at[pl.ds(base + j * ch, ch)], idxb[sn]
                    )
                    @pl.when(j >= nbuf)
                    def _():
                        pltpu.make_async_copy(
                            valb[sn],
                            out_hbm.at[pl.ds(base + (j - nbuf) * ch, ch)],
                            osem[sn],
                        ).wait()

                    pltpu.async_copy(table_hbm.at[idxb[sn]], valb[sn], gsem[sn])

                pltpu.make_async_copy(
                    table_hbm.at[idxb[b]], valb[b], gsem[b]
                ).wait()
                pltpu.async_copy(
                    valb[b], out_hbm.at[pl.ds(base + it * ch, ch)], osem[b]
                )

        for j in range(nbuf):
            r = n_it - nbuf + j
            pltpu.make_async_copy(
                valb[r % nbuf],
                out_hbm.at[pl.ds(base + jnp.int32(r * ch), ch)],
                osem[r % nbuf],
            ).wait()

    return k(table_1d, idx_flat)


def _mlp_body(x_ref, w1_ref, b1_ref, w2_ref, b2_ref, w3_ref, b3_ref, o_ref):
    x = x_ref[...]
    h = jnp.dot(w1_ref[...], x, preferred_element_type=jnp.float32) + b1_ref[...]
    h = jnp.maximum(h, 0.0)
    h = jnp.dot(w2_ref[...], h, preferred_element_type=jnp.float32) + b2_ref[...]
    h = jnp.maximum(h, 0.0)
    o = jnp.dot(w3_ref[...], h, preferred_element_type=jnp.float32) + b3_ref[...]
    sig = 1.0 / (1.0 + jnp.exp(-o))
    sp = jnp.maximum(o, 0.0) + jnp.log(1.0 + jnp.exp(-jnp.abs(o)))
    row = lax.broadcasted_iota(jnp.int32, o.shape, 0)
    o_ref[...] = jnp.where(row < 3, sig, sp)


def _mlp(feats, W1t, b1, W2t, b2, W3t, b3, n):
    blk = 8192
    grid = n // blk
    full = lambda a: pl.BlockSpec(a.shape, lambda i: tuple(i * 0 for _ in a.shape))
    return pl.pallas_call(
        _mlp_body,
        grid=(grid,),
        in_specs=[
            pl.BlockSpec((MLP_IN, blk), lambda i: (i * 0, i)),
            full(W1t), full(b1), full(W2t), full(b2), full(W3t), full(b3),
        ],
        out_specs=pl.BlockSpec((4, blk), lambda i: (i * 0, i)),
        out_shape=jax.ShapeDtypeStruct((4, n), jnp.float32),
    )(feats, W1t, b1, W2t, b2, W3t, b3)


def kernel(positions, tables, W1, b1, W2, b2, W3, b3):
    f32 = jnp.float32
    positions = positions.astype(f32)
    tables = tables.astype(f32)
    W1, b1, W2, b2, W3, b3 = (a.astype(f32) for a in (W1, b1, W2, b2, W3, b3))

    table_1d = (
        tables.reshape(NUM_LEVELS, TABLE_SIZE // 128, 128, F_PER_LEVEL)
        .transpose(0, 1, 3, 2)
        .reshape(-1)
    )
    pos_t = positions.T
    w_args = (W1.T, b1.reshape(HIDDEN, 1), W2.T, b2.reshape(HIDDEN, 1),
              W3.T, b3.reshape(4, 1))
    ns = _NSLICE
    bs = BATCH // ns
    ys = []
    for s in range(ns):
        sl = slice(s * bs, (s + 1) * bs)
        idx_s = _compute_indices(pos_t[:, sl], bs)
        idx_flat = (
            idx_s.reshape(4, 8, bs // 128, 128).transpose(0, 2, 1, 3).reshape(-1)
        )
        out_flat = _gather_sc(table_1d, idx_flat, MLP_IN * bs)
        feats_s = (
            out_flat.reshape(4, bs // 128, 8, 128)
            .transpose(0, 2, 1, 3)
            .reshape(MLP_IN, bs)
        )
        ys.append(_mlp(feats_s, *w_args, bs))
    y = jnp.concatenate(ys, axis=1) if ns > 1 else ys[0]
    yt = y.T.astype(jnp.float64)
    return yt[:, :3], yt[:, 3:4]
